# Initial kernel scaffold; baseline (speedup 1.0000x reference)
import functools

import jax
import jax.numpy as jnp
from jax import lax
from jax.experimental import pallas as pl
from jax.experimental.pallas import tpu as pltpu
from jax.experimental.pallas import tpu_sc as plsc

N = 10000
E = 320000
F = 128
FE = 144
NC = 2
NS = 16
NW = NC * NS
L = 16
EPT = E // NW
B = 80
NB = EPT // B
RPT = N // NS
ROWB = 500
GRID = N // ROWB

_f32 = jnp.float32
_i32 = jnp.int32


def _tc_pre_body(x_ref, w1_ref, b1_ref, w2_ref, b2_ref, wl_ref, wr_ref,
                 xl_ref, xr_ref, xle_ref):
    x = x_ref[...]
    h = jnp.maximum(jnp.dot(x, w1_ref[...], preferred_element_type=_f32)
                    + b1_ref[...], 0.0)
    h = jnp.maximum(jnp.dot(h, w2_ref[...], preferred_element_type=_f32)
                    + b2_ref[...], 0.0)
    xl = jnp.dot(h, wl_ref[...], preferred_element_type=_f32)
    xr = jnp.dot(h, wr_ref[...], preferred_element_type=_f32)
    xl_ref[...] = xl
    xr_ref[...] = xr
    ones = jnp.where(lax.broadcasted_iota(_i32, (ROWB, FE - F), 1) == 0,
                     1.0, 0.0).astype(_f32)
    xle_ref[...] = jnp.concatenate([xl, ones], axis=1)


def _tc_pre(x, w1, b1, w2, b2, wl, wr):
    blk = lambda r, c: pl.BlockSpec((r, c), lambda i: (0, 0))
    return pl.pallas_call(
        _tc_pre_body,
        grid=(GRID,),
        in_specs=[
            pl.BlockSpec((ROWB, F), lambda i: (i, 0)),
            blk(F, F), blk(1, F), blk(F, F), blk(1, F), blk(F, F), blk(F, F),
        ],
        out_specs=[
            pl.BlockSpec((ROWB, F), lambda i: (i, 0)),
            pl.BlockSpec((ROWB, F), lambda i: (i, 0)),
            pl.BlockSpec((ROWB, FE), lambda i: (i, 0)),
        ],
        out_shape=[
            jax.ShapeDtypeStruct((N, F), _f32),
            jax.ShapeDtypeStruct((N, F), _f32),
            jax.ShapeDtypeStruct((N, FE), _f32),
        ],
    )(x, w1, b1, w2, b2, wl, wr)


def _sc_logits_body(xl_hbm, xr_hbm, att_hbm, ei_hbm,
                    logits_hbm, tmax_hbm,
                    src2, dst2, sbuf, att_v, logit_v, maxbuf, sem):
    cid = lax.axis_index("c")
    sid = lax.axis_index("s")
    wid = sid * NC + cid
    pltpu.sync_copy(ei_hbm.at[0, wid], src2)
    pltpu.sync_copy(ei_hbm.at[1, wid], dst2)
    pltpu.sync_copy(att_hbm, att_v)

    def jbody(j, _):
        pltpu.async_copy(xl_hbm.at[src2.at[j]], sbuf, sem).wait()
        pltpu.async_copy(xr_hbm.at[dst2.at[j]], sbuf, sem, add=True).wait()

        def gbody(g, _):
            rows = lax.broadcasted_iota(_i32, (L,), 0) + g * L
            accs = [jnp.zeros((L,), _f32) for _ in range(4)]
            for f in range(F):
                colv = jnp.full((L,), f, _i32)
                v = plsc.load_gather(sbuf, [rows, colv])
                a = att_v[f]
                av = jnp.zeros((L,), _f32) + a
                accs[f % 4] = accs[f % 4] + av * jnp.maximum(v, 0.2 * v)
            acc = (accs[0] + accs[1]) + (accs[2] + accs[3])
            logit_v[pl.ds(j * B + g * L, L)] = acc
            return 0

        lax.fori_loop(0, B // L, gbody, 0)
        return 0

    lax.fori_loop(0, NB, jbody, 0)

    def mbody(i, m):
        return jnp.maximum(m, logit_v[pl.ds(i * L, L)])

    m = lax.fori_loop(0, EPT // L, mbody, jnp.full((L,), -3.4e38, _f32))
    maxbuf[...] = m
    pltpu.sync_copy(maxbuf, tmax_hbm.at[pl.ds(wid * L, L)])
    pltpu.sync_copy(logit_v, logits_hbm.at[wid])


def _sc_logits(xl, xr, att, ei):
    mesh = plsc.VectorSubcoreMesh(core_axis_name="c", subcore_axis_name="s")
    fn = pl.kernel(
        _sc_logits_body,
        out_type=[
            jax.ShapeDtypeStruct((NW, EPT), _f32),
            jax.ShapeDtypeStruct((NW * L,), _f32),
        ],
        mesh=mesh,
        scratch_types=[
            pltpu.VMEM((NB, B), _i32),
            pltpu.VMEM((NB, B), _i32),
            pltpu.VMEM((B, F), _f32),
            pltpu.VMEM((F,), _f32),
            pltpu.VMEM((EPT,), _f32),
            pltpu.VMEM((L,), _f32),
            pltpu.SemaphoreType.DMA,
        ],
    )
    return fn(xl, xr, att, ei)


def _sc_aggr_body(xle_hbm, ei_hbm, logits_hbm, tmax_hbm, zrows_hbm,
                  u_hbm,
                  src2, dst2, rowbuf, ex_v, tmax_v, u_sp, sem):
    cid = lax.axis_index("c")
    sid = lax.axis_index("s")
    wid = sid * NC + cid
    pltpu.sync_copy(zrows_hbm, u_sp.at[pl.ds(sid * RPT, RPT)])
    pltpu.sync_copy(ei_hbm.at[0, wid], src2)
    pltpu.sync_copy(ei_hbm.at[1, wid], dst2)
    pltpu.sync_copy(logits_hbm.at[wid], ex_v)
    pltpu.sync_copy(tmax_hbm, tmax_v)

    def mb(i, m):
        return jnp.maximum(m, tmax_v[pl.ds(i * L, L)])

    m = lax.fori_loop(0, NW, mb, jnp.full((L,), -3.4e38, _f32))
    gmax = jnp.max(m)

    def eb(i, _):
        v = ex_v[pl.ds(i * L, L)]
        ex_v[pl.ds(i * L, L)] = jnp.exp(v - gmax)
        return 0

    lax.fori_loop(0, EPT // L, eb, 0)
    plsc.subcore_barrier()

    def jbody(j, _):
        pltpu.async_copy(xle_hbm.at[src2.at[j]], rowbuf, sem).wait()

        def ebody(e, _):
            s = ex_v[j * B + e]
            for c in range(FE // L):
                sl = rowbuf[e, pl.ds(c * L, L)]
                rowbuf[e, pl.ds(c * L, L)] = sl * s
            return 0

        lax.fori_loop(0, B, ebody, 0)
        pltpu.sync_copy(rowbuf, u_sp.at[dst2.at[j]], add=True)
        return 0

    lax.fori_loop(0, NB, jbody, 0)
    plsc.subcore_barrier()
    pltpu.sync_copy(u_sp.at[pl.ds(sid * RPT, RPT)],
                    u_hbm.at[cid, pl.ds(sid * RPT, RPT)])


def _sc_aggr(xle, ei, logits, tmax, zrows):
    mesh = plsc.VectorSubcoreMesh(core_axis_name="c", subcore_axis_name="s")
    fn = pl.kernel(
        _sc_aggr_body,
        out_type=jax.ShapeDtypeStruct((NC, N, FE), _f32),
        mesh=mesh,
        scratch_types=[
            pltpu.VMEM((NB, B), _i32),
            pltpu.VMEM((NB, B), _i32),
            pltpu.VMEM((B, FE), _f32),
            pltpu.VMEM((EPT,), _f32),
            pltpu.VMEM((NW * L,), _f32),
            pltpu.VMEM_SHARED((N, FE), _f32),
            pltpu.SemaphoreType.DMA,
        ],
    )
    return fn(xle, ei, logits, tmax, zrows)


def _tc_post_body(u_ref, h_ref, gb_ref, wih_ref, bih_ref, whh_ref, bhh_ref,
                  wq_ref, bq_ref, q_ref, hout_ref):
    u = u_ref[0] + u_ref[1]
    den = jnp.sum(u[:, F:FE], axis=1, keepdims=True)
    gat = jnp.maximum(u[:, :F] / (den + 1e-16) + gb_ref[...], 0.0)
    h_in = h_ref[...]
    gi = jnp.dot(gat, wih_ref[...], preferred_element_type=_f32) + bih_ref[...]
    gh = jnp.dot(h_in, whh_ref[...], preferred_element_type=_f32) + bhh_ref[...]
    r = jax.nn.sigmoid(gi[:, :F] + gh[:, :F])
    z = jax.nn.sigmoid(gi[:, F:2 * F] + gh[:, F:2 * F])
    n = jnp.tanh(gi[:, 2 * F:] + r * gh[:, 2 * F:])
    h = (1.0 - z) * n + z * h_in
    q_ref[...] = jnp.dot(h, wq_ref[...], preferred_element_type=_f32) + bq_ref[...]
    hout_ref[...] = h


def _tc_post(u, hidden, gbias, wih, bih, whh, bhh, wq, bq):
    blk = lambda r, c: pl.BlockSpec((r, c), lambda i: (0, 0))
    return pl.pallas_call(
        _tc_post_body,
        grid=(GRID,),
        in_specs=[
            pl.BlockSpec((NC, ROWB, FE), lambda i: (0, i, 0)),
            pl.BlockSpec((ROWB, F), lambda i: (i, 0)),
            blk(1, F),
            blk(F, 3 * F), blk(1, 3 * F), blk(F, 3 * F), blk(1, 3 * F),
            blk(F, 16), blk(1, 16),
        ],
        out_specs=[
            pl.BlockSpec((ROWB, 16), lambda i: (i, 0)),
            pl.BlockSpec((ROWB, F), lambda i: (i, 0)),
        ],
        out_shape=[
            jax.ShapeDtypeStruct((N, 16), _f32),
            jax.ShapeDtypeStruct((N, F), _f32),
        ],
    )(u, hidden, gbias, wih, bih, whh, bhh, wq, bq)


def kernel(inputs, hidden_states, W1, b1, W2, b2, Wl, Wr, att, gbias,
           Wih, Whh, bih, bhh, Wq, bq, edge_index):
    xl, xr, xle = _tc_pre(inputs, W1, b1.reshape(1, F), W2, b2.reshape(1, F),
                          Wl, Wr)
    ei = edge_index.reshape(2, NW, NB, B)
    logits, tmax = _sc_logits(xl, xr, att, ei)
    zrows = jnp.zeros((RPT, FE), _f32)
    u = _sc_aggr(xle, ei, logits, tmax, zrows)
    q, h = _tc_post(u, hidden_states, gbias.reshape(1, F),
                    Wih, bih.reshape(1, 3 * F), Whh, bhh.reshape(1, 3 * F),
                    Wq, bq.reshape(1, 16))
    return (q, h)


# speedup vs baseline: 6.9544x; 6.9544x over previous
import jax
import jax.numpy as jnp
from jax import lax
from jax.experimental import pallas as pl
from jax.experimental.pallas import tpu as pltpu
from jax.experimental.pallas import tpu_sc as plsc

N = 10000
E = 320000
F = 128
NC = 2
NS = 16
NW = NC * NS
L = 16
EPT = E // NW
B = 80
NB = EPT // B
NR = NW * NB
NP = 10240
RPT = NP // NS
NPG = NP // 8
GPT = NPG // NS
ZR = 80
ROWB = 1000
GRID = N // ROWB
ROWB2 = 1024
GRID2 = NP // ROWB2

_f32 = jnp.float32
_i32 = jnp.int32


def _tc_pre_body(x_ref, w1_ref, b1_ref, w2_ref, b2_ref, wl_ref, wr_ref,
                 xl_ref, xr_ref):
    x = x_ref[...]
    h = jnp.maximum(jnp.dot(x, w1_ref[...], preferred_element_type=_f32)
                    + b1_ref[...], 0.0)
    h = jnp.maximum(jnp.dot(h, w2_ref[...], preferred_element_type=_f32)
                    + b2_ref[...], 0.0)
    xl_ref[...] = jnp.dot(h, wl_ref[...], preferred_element_type=_f32)
    xr_ref[...] = jnp.dot(h, wr_ref[...], preferred_element_type=_f32)


def _tc_pre(x, w1, b1, w2, b2, wl, wr):
    blk = lambda r, c: pl.BlockSpec((r, c), lambda i: (0, 0))
    return pl.pallas_call(
        _tc_pre_body,
        grid=(GRID,),
        in_specs=[
            pl.BlockSpec((ROWB, F), lambda i: (i, 0)),
            blk(F, F), blk(1, F), blk(F, F), blk(1, F), blk(F, F), blk(F, F),
        ],
        out_specs=[
            pl.BlockSpec((ROWB, F), lambda i: (i, 0)),
            pl.BlockSpec((ROWB, F), lambda i: (i, 0)),
        ],
        out_shape=[
            jax.ShapeDtypeStruct((N, F), _f32),
            jax.ShapeDtypeStruct((N, F), _f32),
        ],
    )(x, w1, b1, w2, b2, wl, wr)


def _load_chunks(wid, srcs_dsts):
    for table, dst in srcs_dsts:
        pltpu.sync_copy(table.at[wid], dst)


def _sc_logits_body(xl_hbm, xr_hbm, att_hbm, src_hbm, dst_hbm,
                    logits_hbm, tmax_hbm,
                    src2, dst2, sbuf, att_v, logit_v, maxbuf, sem):
    cid = lax.axis_index("c")
    sid = lax.axis_index("s")
    wid = sid * NC + cid
    _load_chunks(wid, [(src_hbm, src2), (dst_hbm, dst2)])
    pltpu.sync_copy(att_hbm, att_v)
    att_chunks = [att_v[pl.ds(c * L, L)] for c in range(F // L)]

    lane = lax.broadcasted_iota(_i32, (L,), 0)
    perms = [jnp.bitwise_xor(lane, sh) for sh in (8, 4, 2, 1)]

    def lanesum(v):
        for p in perms:
            v = v + v.at[p].get(mode="promise_in_bounds", unique_indices=True)
        return v

    def jbody(j, _):
        pltpu.async_copy(xl_hbm.at[src2.at[j]], sbuf, sem).wait()
        pltpu.async_copy(xr_hbm.at[dst2.at[j]], sbuf, sem, add=True).wait()

        def gbody(g, _):
            r = jnp.zeros((L,), _f32)
            for k in range(L):
                e = g * L + k
                accs = [jnp.zeros((L,), _f32) for _ in range(4)]
                for c in range(F // L):
                    v = sbuf[e, pl.ds(c * L, L)]
                    accs[c % 4] = (accs[c % 4]
                                   + att_chunks[c] * jnp.maximum(v, 0.2 * v))
                acc = (accs[0] + accs[1]) + (accs[2] + accs[3])
                r = jnp.where(lane == k, lanesum(acc), r)
            logit_v[j, pl.ds(g * L, L)] = r
            return 0

        lax.fori_loop(0, B // L, gbody, 0)
        return 0

    lax.fori_loop(0, NB, jbody, 0)

    def mbody(j, m):
        for c in range(B // L):
            m = jnp.maximum(m, logit_v[j, pl.ds(c * L, L)])
        return m

    m = lax.fori_loop(0, NB, mbody, jnp.full((L,), -3.4e38, _f32))
    maxbuf[...] = m
    pltpu.sync_copy(maxbuf, tmax_hbm.at[pl.ds(wid * L, L)])
    pltpu.sync_copy(logit_v, logits_hbm.at[wid])


def _sc_logits(xl, xr, att, src, dst):
    mesh = plsc.VectorSubcoreMesh(core_axis_name="c", subcore_axis_name="s")
    fn = pl.kernel(
        _sc_logits_body,
        out_type=[
            jax.ShapeDtypeStruct((NW, NB, B), _f32),
            jax.ShapeDtypeStruct((NW * L,), _f32),
        ],
        mesh=mesh,
        scratch_types=[
            pltpu.VMEM((NB, B), _i32),
            pltpu.VMEM((NB, B), _i32),
            pltpu.VMEM((B, F), _f32),
            pltpu.VMEM((F,), _f32),
            pltpu.VMEM((NB, B), _f32),
            pltpu.VMEM((L,), _f32),
            pltpu.SemaphoreType.DMA,
        ],
    )
    return fn(xl, xr, att, src, dst)


def _sc_aggr_body(xl_hbm, src_hbm, dst_hbm, logits_hbm, tmax_hbm, zrows_hbm,
                  u_hbm, den_hbm,
                  srcrow, dstrow, dstgrow, lrow, rowbuf, denbuf, tmax_v,
                  u_sp, den_sp, sem):
    cid = lax.axis_index("c")
    sid = lax.axis_index("s")
    wid = sid * NC + cid
    for t in range(RPT // ZR):
        pltpu.sync_copy(zrows_hbm, u_sp.at[pl.ds(sid * RPT + t * ZR, ZR)])
    pltpu.sync_copy(zrows_hbm, den_sp.at[pl.ds(sid * GPT, GPT)])
    pltpu.sync_copy(tmax_hbm, tmax_v)

    def mb(i, m):
        return jnp.maximum(m, tmax_v[pl.ds(i * L, L)])

    m = lax.fori_loop(0, NW, mb, jnp.full((L,), -3.4e38, _f32))
    lane = lax.broadcasted_iota(_i32, (L,), 0)
    for sh in (8, 4, 2, 1):
        p = jnp.bitwise_xor(lane, sh)
        m = jnp.maximum(m, m.at[p].get(mode="promise_in_bounds",
                                       unique_indices=True))
    gmax = m
    plsc.subcore_barrier()

    onesv = jnp.full((L,), 1.0, _f32)

    def jbody(j, _):
        pltpu.sync_copy(src_hbm.at[wid, j], srcrow)
        pltpu.sync_copy(dst_hbm.at[wid, j], dstrow)
        pltpu.sync_copy(logits_hbm.at[wid, j], lrow)
        pltpu.async_copy(xl_hbm.at[srcrow.at[0]], rowbuf, sem).wait()
        for c in range(B // L):
            dstgrow[0, pl.ds(c * L, L)] = dstrow[0, pl.ds(c * L, L)] >> 3
            v = lrow[0, pl.ds(c * L, L)]
            lrow[0, pl.ds(c * L, L)] = jnp.exp(v - gmax)

        def gbody(g, _):
            exvec = lrow[0, pl.ds(g * L, L)]
            dstv = dstrow[0, pl.ds(g * L, L)]
            for k in range(L):
                s = exvec[k]
                e = g * L + k
                for c in range(F // L):
                    rowbuf[e, pl.ds(c * L, L)] = rowbuf[e, pl.ds(c * L, L)] * s
                dmod = dstv[k] & 7
                for c in range(F // L):
                    coef = (dmod == c).astype(_f32)
                    denbuf[e, pl.ds(c * L, L)] = onesv * (s * coef)
            return 0

        lax.fori_loop(0, B // L, gbody, 0)
        pltpu.sync_copy(rowbuf, u_sp.at[dstrow.at[0]], add=True)
        pltpu.sync_copy(denbuf, den_sp.at[dstgrow.at[0]], add=True)
        return 0

    lax.fori_loop(0, NB, jbody, 0)
    plsc.subcore_barrier()
    pltpu.sync_copy(u_sp.at[pl.ds(sid * RPT, RPT)],
                    u_hbm.at[cid, pl.ds(sid * RPT, RPT)])
    pltpu.sync_copy(den_sp.at[pl.ds(sid * GPT, GPT)],
                    den_hbm.at[cid, pl.ds(sid * GPT, GPT)])


def _sc_aggr(xl, src, dst, logits, tmax, zrows):
    mesh = plsc.VectorSubcoreMesh(core_axis_name="c", subcore_axis_name="s")
    fn = pl.kernel(
        _sc_aggr_body,
        out_type=[
            jax.ShapeDtypeStruct((NC, NP, F), _f32),
            jax.ShapeDtypeStruct((NC, NPG, F), _f32),
        ],
        mesh=mesh,
        scratch_types=[
            pltpu.VMEM((1, B), _i32),
            pltpu.VMEM((1, B), _i32),
            pltpu.VMEM((1, B), _i32),
            pltpu.VMEM((1, B), _f32),
            pltpu.VMEM((B, F), _f32),
            pltpu.VMEM((B, F), _f32),
            pltpu.VMEM((NW * L,), _f32),
            pltpu.VMEM_SHARED((NP, F), _f32),
            pltpu.VMEM_SHARED((NPG, F), _f32),
            pltpu.SemaphoreType.DMA,
        ],
    )
    return fn(xl, src, dst, logits, tmax, zrows)


def _tc_post_body(u_ref, den_ref, h_ref, gb_ref, wih_ref, bih_ref,
                  whh_ref, bhh_ref, wq_ref, bq_ref, q_ref, hout_ref):
    u = u_ref[0] + u_ref[1]
    den = den_ref[...]
    gat = jnp.maximum(u / (den + 1e-16) + gb_ref[...], 0.0)
    h_in = h_ref[...]
    gi = jnp.dot(gat, wih_ref[...], preferred_element_type=_f32) + bih_ref[...]
    gh = jnp.dot(h_in, whh_ref[...], preferred_element_type=_f32) + bhh_ref[...]
    r = jax.nn.sigmoid(gi[:, :F] + gh[:, :F])
    z = jax.nn.sigmoid(gi[:, F:2 * F] + gh[:, F:2 * F])
    n = jnp.tanh(gi[:, 2 * F:] + r * gh[:, 2 * F:])
    h = (1.0 - z) * n + z * h_in
    q_ref[...] = jnp.dot(h, wq_ref[...], preferred_element_type=_f32) + bq_ref[...]
    hout_ref[...] = h


def _tc_post(u, den, hidden, gbias, wih, bih, whh, bhh, wq, bq):
    blk = lambda r, c: pl.BlockSpec((r, c), lambda i: (0, 0))
    return pl.pallas_call(
        _tc_post_body,
        grid=(GRID2,),
        in_specs=[
            pl.BlockSpec((NC, ROWB2, F), lambda i: (0, i, 0)),
            pl.BlockSpec((ROWB2, 1), lambda i: (i, 0)),
            pl.BlockSpec((ROWB2, F), lambda i: (i, 0)),
            blk(1, F),
            blk(F, 3 * F), blk(1, 3 * F), blk(F, 3 * F), blk(1, 3 * F),
            blk(F, 16), blk(1, 16),
        ],
        out_specs=[
            pl.BlockSpec((ROWB2, 16), lambda i: (i, 0)),
            pl.BlockSpec((ROWB2, F), lambda i: (i, 0)),
        ],
        out_shape=[
            jax.ShapeDtypeStruct((NP, 16), _f32),
            jax.ShapeDtypeStruct((NP, F), _f32),
        ],
    )(u, den, hidden, gbias, wih, bih, whh, bhh, wq, bq)


def kernel(inputs, hidden_states, W1, b1, W2, b2, Wl, Wr, att, gbias,
           Wih, Whh, bih, bhh, Wq, bq, edge_index):
    xl, xr = _tc_pre(inputs, W1, b1.reshape(1, F), W2, b2.reshape(1, F),
                     Wl, Wr)
    src = edge_index[0].reshape(NW, NB, B)
    dst = edge_index[1].reshape(NW, NB, B)
    logits, tmax = _sc_logits(xl, xr, att, src, dst)
    zrows = jnp.zeros((ZR, F), _f32)
    u, den_raw = _sc_aggr(xl, src.reshape(NW, NB, 1, B), dst.reshape(NW, NB, 1, B), logits.reshape(NW, NB, 1, B), tmax, zrows)
    den = (den_raw[0] + den_raw[1]).reshape(NPG, 8, L)[:, :, 0].reshape(NP, 1)
    hidden_p = jnp.pad(hidden_states, ((0, NP - N), (0, 0)))
    q, h = _tc_post(u, den, hidden_p, gbias.reshape(1, F),
                    Wih, bih.reshape(1, 3 * F), Whh, bhh.reshape(1, 3 * F),
                    Wq, bq.reshape(1, 16))
    return (q[:N], h[:N])


# speedup vs baseline: 13.6188x; 1.9583x over previous
import jax
import jax.numpy as jnp
from jax import lax
from jax.experimental import pallas as pl
from jax.experimental.pallas import tpu as pltpu
from jax.experimental.pallas import tpu_sc as plsc

N = 10000
E = 320000
F = 128
NC = 2
NS = 16
NW = NC * NS
L = 16
EPT = E // NW
B = 80
NB = EPT // B
NT = (NB - 1) // 2
NP = 10240
RPT = NP // NS
DPT = NP // NS
ZR = 80
ROWB = 1000
GRID = N // ROWB
ROWB2 = 1024
GRID2 = NP // ROWB2

_f32 = jnp.float32
_i32 = jnp.int32


def _tc_pre_body(x_ref, w1_ref, b1_ref, w2_ref, b2_ref, wl_ref, wr_ref,
                 xl_ref, xr_ref):
    x = x_ref[...]
    h = jnp.maximum(jnp.dot(x, w1_ref[...], preferred_element_type=_f32)
                    + b1_ref[...], 0.0)
    h = jnp.maximum(jnp.dot(h, w2_ref[...], preferred_element_type=_f32)
                    + b2_ref[...], 0.0)
    xl_ref[...] = jnp.dot(h, wl_ref[...], preferred_element_type=_f32)
    xr_ref[...] = jnp.dot(h, wr_ref[...], preferred_element_type=_f32)


def _tc_pre(x, w1, b1, w2, b2, wl, wr):
    blk = lambda r, c: pl.BlockSpec((r, c), lambda i: (0, 0))
    return pl.pallas_call(
        _tc_pre_body,
        grid=(GRID,),
        in_specs=[
            pl.BlockSpec((ROWB, F), lambda i: (i, 0)),
            blk(F, F), blk(1, F), blk(F, F), blk(1, F), blk(F, F), blk(F, F),
        ],
        out_specs=[
            pl.BlockSpec((ROWB, F), lambda i: (i, 0)),
            pl.BlockSpec((ROWB, F), lambda i: (i, 0)),
        ],
        out_shape=[
            jax.ShapeDtypeStruct((N, F), _f32),
            jax.ShapeDtypeStruct((N, F), _f32),
        ],
    )(x, w1, b1, w2, b2, wl, wr)


def _sc_logits_body(xl_hbm, xr_hbm, att_hbm, src_hbm, dst_hbm,
                    logits_hbm, tmax_hbm,
                    src2, dst2, sbuf0, sbuf1, att_v, logit_v, maxbuf,
                    semA0, semA1, semB0, semB1):
    cid = lax.axis_index("c")
    sid = lax.axis_index("s")
    wid = sid * NC + cid
    pltpu.sync_copy(src_hbm.at[wid], src2)
    pltpu.sync_copy(dst_hbm.at[wid], dst2)
    pltpu.sync_copy(att_hbm, att_v)
    att_chunks = [att_v[pl.ds(c * L, L)] for c in range(F // L)]

    lane = lax.broadcasted_iota(_i32, (L,), 0)
    perms = [jnp.bitwise_xor(lane, sh) for sh in (8, 4, 2, 1)]

    def lanesum(v):
        for p in perms:
            v = v + v.at[p].get(mode="promise_in_bounds", unique_indices=True)
        return v

    def issue_xl(j, buf, sem):
        pltpu.async_copy(xl_hbm.at[src2.at[j]], buf, sem)

    def wait_gather(buf, sem):
        pltpu.make_async_copy(xl_hbm.at[src2.at[0]], buf, sem).wait()

    def issue_xr(j, buf, sem):
        pltpu.async_copy(xr_hbm.at[dst2.at[j]], buf, sem, add=True)

    def compute(j, buf):
        def gbody(g, _):
            r = jnp.zeros((L,), _f32)
            for k in range(L):
                e = g * L + k
                accs = [jnp.zeros((L,), _f32) for _ in range(4)]
                for c in range(F // L):
                    v = buf[e, pl.ds(c * L, L)]
                    accs[c % 4] = (accs[c % 4]
                                   + att_chunks[c] * jnp.maximum(v, 0.2 * v))
                acc = (accs[0] + accs[1]) + (accs[2] + accs[3])
                r = jnp.where(lane == k, lanesum(acc), r)
            logit_v[j, pl.ds(g * L, L)] = r
            return 0

        lax.fori_loop(0, B // L, gbody, 0)

    issue_xl(0, sbuf0, semA0)

    def body(t, _):
        j0 = 2 * t
        j1 = j0 + 1
        issue_xl(j1, sbuf1, semA1)
        wait_gather(sbuf0, semA0)
        issue_xr(j0, sbuf0, semB0)
        wait_gather(sbuf1, semA1)
        issue_xr(j1, sbuf1, semB1)
        wait_gather(sbuf0, semB0)
        compute(j0, sbuf0)
        issue_xl(j0 + 2, sbuf0, semA0)
        wait_gather(sbuf1, semB1)
        compute(j1, sbuf1)
        return 0

    lax.fori_loop(0, NT, body, 0)
    wait_gather(sbuf0, semA0)
    issue_xr(NB - 1, sbuf0, semB0)
    wait_gather(sbuf0, semB0)
    compute(NB - 1, sbuf0)

    def mbody(j, m):
        for c in range(B // L):
            m = jnp.maximum(m, logit_v[j, pl.ds(c * L, L)])
        return m

    m = lax.fori_loop(0, NB, mbody, jnp.full((L,), -3.4e38, _f32))
    maxbuf[...] = m
    pltpu.sync_copy(maxbuf, tmax_hbm.at[pl.ds(wid * L, L)])
    pltpu.sync_copy(logit_v, logits_hbm.at[wid])


def _sc_logits(xl, xr, att, src, dst):
    mesh = plsc.VectorSubcoreMesh(core_axis_name="c", subcore_axis_name="s")
    fn = pl.kernel(
        _sc_logits_body,
        out_type=[
            jax.ShapeDtypeStruct((NW, NB, B), _f32),
            jax.ShapeDtypeStruct((NW * L,), _f32),
        ],
        mesh=mesh,
        scratch_types=[
            pltpu.VMEM((NB, B), _i32),
            pltpu.VMEM((NB, B), _i32),
            pltpu.VMEM((B, F), _f32),
            pltpu.VMEM((B, F), _f32),
            pltpu.VMEM((F,), _f32),
            pltpu.VMEM((NB, B), _f32),
            pltpu.VMEM((L,), _f32),
            pltpu.SemaphoreType.DMA,
            pltpu.SemaphoreType.DMA,
            pltpu.SemaphoreType.DMA,
            pltpu.SemaphoreType.DMA,
        ],
    )
    return fn(xl, xr, att, src, dst)


def _sc_aggr_body(xl_hbm, packed_hbm, lg_hbm, tmax_hbm, zrows_hbm, zvec_hbm,
                  u_hbm, den_hbm,
                  prow0, prow1, lrow0, lrow1, exrow, rowbuf0, rowbuf1, tmax_v,
                  u_sp, den_sp,
                  semP, semP2, semL, semL2, semg0, semg1, semS):
    cid = lax.axis_index("c")
    sid = lax.axis_index("s")
    wid = sid * NC + cid
    for t in range(RPT // ZR):
        pltpu.sync_copy(zrows_hbm, u_sp.at[pl.ds(sid * RPT + t * ZR, ZR)])
    pltpu.sync_copy(zvec_hbm, den_sp.at[pl.ds(sid * DPT, DPT)])
    pltpu.sync_copy(tmax_hbm, tmax_v)

    def mb(i, m):
        return jnp.maximum(m, tmax_v[pl.ds(i * L, L)])

    m = lax.fori_loop(0, NW, mb, jnp.full((L,), -3.4e38, _f32))
    lane = lax.broadcasted_iota(_i32, (L,), 0)
    for sh in (8, 4, 2, 1):
        p = jnp.bitwise_xor(lane, sh)
        m = jnp.maximum(m, m.at[p].get(mode="promise_in_bounds",
                                       unique_indices=True))
    gmax = m
    plsc.subcore_barrier()

    def issue_rows(j, prow, sem):
        pltpu.async_copy(packed_hbm.at[wid, j], prow, sem)

    def wait_rows(prow, sem):
        pltpu.make_async_copy(packed_hbm.at[wid, 0], prow, sem).wait()

    def issue_lrow(j, lrow, sem):
        pltpu.async_copy(lg_hbm.at[wid, j], lrow, sem)

    def wait_lrow(lrow, sem):
        pltpu.make_async_copy(lg_hbm.at[wid, 0], lrow, sem).wait()

    def issue_gather(prow, buf, sem):
        pltpu.async_copy(xl_hbm.at[prow.at[0]], buf, sem)

    def wait_gather(buf, sem):
        pltpu.make_async_copy(xl_hbm.at[prow0.at[0]], buf, sem).wait()

    def compute(prow, lrow, buf):
        for c in range(B // L):
            lv = lrow[0, pl.ds(c * L, L)]
            exrow[pl.ds(c * L, L)] = jnp.exp(lv - gmax)

        def gbody(g, _):
            exvec = exrow[pl.ds(g * L, L)]
            for k in range(L):
                s = exvec[k]
                e = g * L + k
                for c in range(F // L):
                    buf[e, pl.ds(c * L, L)] = buf[e, pl.ds(c * L, L)] * s
            return 0

        lax.fori_loop(0, B // L, gbody, 0)
        cpu_ = pltpu.async_copy(buf, u_sp.at[prow.at[1]], semS, add=True)
        cpe_ = pltpu.async_copy(exrow, den_sp.at[prow.at[1]], semS, add=True)
        cpu_.wait()
        cpe_.wait()

    pltpu.sync_copy(packed_hbm.at[wid, 0], prow0)
    pltpu.sync_copy(lg_hbm.at[wid, 0], lrow0)
    issue_rows(1, prow1, semP)
    issue_lrow(1, lrow1, semL)
    issue_gather(prow0, rowbuf0, semg0)

    def body(t, _):
        j0 = 2 * t
        j1 = j0 + 1
        wait_rows(prow1, semP)
        issue_gather(prow1, rowbuf1, semg1)
        wait_gather(rowbuf0, semg0)
        compute(prow0, lrow0, rowbuf0)
        issue_rows(j0 + 2, prow0, semP2)
        issue_lrow(j0 + 2, lrow0, semL2)
        wait_gather(rowbuf1, semg1)
        wait_lrow(lrow1, semL)
        compute(prow1, lrow1, rowbuf1)
        issue_rows(j1 + 2, prow1, semP)
        issue_lrow(j1 + 2, lrow1, semL)
        wait_rows(prow0, semP2)
        wait_lrow(lrow0, semL2)
        issue_gather(prow0, rowbuf0, semg0)
        return 0

    lax.fori_loop(0, NT, body, 0)
    wait_rows(prow1, semP)
    wait_lrow(lrow1, semL)
    wait_gather(rowbuf0, semg0)
    compute(prow0, lrow0, rowbuf0)

    plsc.subcore_barrier()
    pltpu.sync_copy(u_sp.at[pl.ds(sid * RPT, RPT)],
                    u_hbm.at[cid, pl.ds(sid * RPT, RPT)])
    pltpu.sync_copy(den_sp.at[pl.ds(sid * DPT, DPT)],
                    den_hbm.at[cid, pl.ds(sid * DPT, DPT)])


def _sc_aggr(xl, packed, lg, tmax, zrows, zvec):
    mesh = plsc.VectorSubcoreMesh(core_axis_name="c", subcore_axis_name="s")
    fn = pl.kernel(
        _sc_aggr_body,
        out_type=[
            jax.ShapeDtypeStruct((NC, NP, F), _f32),
            jax.ShapeDtypeStruct((NC, NP), _f32),
        ],
        mesh=mesh,
        scratch_types=[
            pltpu.VMEM((2, B), _i32),
            pltpu.VMEM((2, B), _i32),
            pltpu.VMEM((1, B), _f32),
            pltpu.VMEM((1, B), _f32),
            pltpu.VMEM((B,), _f32),
            pltpu.VMEM((B, F), _f32),
            pltpu.VMEM((B, F), _f32),
            pltpu.VMEM((NW * L,), _f32),
            pltpu.VMEM_SHARED((NP, F), _f32),
            pltpu.VMEM_SHARED((NP,), _f32),
            pltpu.SemaphoreType.DMA,
            pltpu.SemaphoreType.DMA,
            pltpu.SemaphoreType.DMA,
            pltpu.SemaphoreType.DMA,
            pltpu.SemaphoreType.DMA,
            pltpu.SemaphoreType.DMA,
            pltpu.SemaphoreType.DMA,
        ],
    )
    return fn(xl, packed, lg, tmax, zrows, zvec)


def _tc_post_body(u_ref, den_ref, h_ref, gb_ref, wih_ref, bih_ref,
                  whh_ref, bhh_ref, wq_ref, bq_ref, q_ref, hout_ref):
    u = u_ref[0] + u_ref[1]
    den = den_ref[0] + den_ref[1]
    gat = jnp.maximum(u / (den + 1e-16) + gb_ref[...], 0.0)
    h_in = h_ref[...]
    gi = jnp.dot(gat, wih_ref[...], preferred_element_type=_f32) + bih_ref[...]
    gh = jnp.dot(h_in, whh_ref[...], preferred_element_type=_f32) + bhh_ref[...]
    r = jax.nn.sigmoid(gi[:, :F] + gh[:, :F])
    z = jax.nn.sigmoid(gi[:, F:2 * F] + gh[:, F:2 * F])
    n = jnp.tanh(gi[:, 2 * F:] + r * gh[:, 2 * F:])
    h = (1.0 - z) * n + z * h_in
    q_ref[...] = jnp.dot(h, wq_ref[...], preferred_element_type=_f32) + bq_ref[...]
    hout_ref[...] = h


def _tc_post(u, den, hidden, gbias, wih, bih, whh, bhh, wq, bq):
    blk = lambda r, c: pl.BlockSpec((r, c), lambda i: (0, 0))
    return pl.pallas_call(
        _tc_post_body,
        grid=(GRID2,),
        in_specs=[
            pl.BlockSpec((NC, ROWB2, F), lambda i: (0, i, 0)),
            pl.BlockSpec((NC, ROWB2, 1), lambda i: (0, i, 0)),
            pl.BlockSpec((ROWB2, F), lambda i: (i, 0)),
            blk(1, F),
            blk(F, 3 * F), blk(1, 3 * F), blk(F, 3 * F), blk(1, 3 * F),
            blk(F, 16), blk(1, 16),
        ],
        out_specs=[
            pl.BlockSpec((ROWB2, 16), lambda i: (i, 0)),
            pl.BlockSpec((ROWB2, F), lambda i: (i, 0)),
        ],
        out_shape=[
            jax.ShapeDtypeStruct((NP, 16), _f32),
            jax.ShapeDtypeStruct((NP, F), _f32),
        ],
    )(u, den, hidden, gbias, wih, bih, whh, bhh, wq, bq)


def kernel(inputs, hidden_states, W1, b1, W2, b2, Wl, Wr, att, gbias,
           Wih, Whh, bih, bhh, Wq, bq, edge_index):
    xl, xr = _tc_pre(inputs, W1, b1.reshape(1, F), W2, b2.reshape(1, F),
                     Wl, Wr)
    src = edge_index[0].reshape(NW, NB, B)
    dst = edge_index[1].reshape(NW, NB, B)
    logits, tmax = _sc_logits(xl, xr, att, src, dst)
    packed = jnp.pad(jnp.stack([src, dst], axis=2),
                     ((0, 0), (0, 1), (0, 0), (0, 0)))
    lg = jnp.pad(logits, ((0, 0), (0, 1), (0, 0))).reshape(NW, NB + 1, 1, B)
    zrows = jnp.zeros((ZR, F), _f32)
    zvec = jnp.zeros((DPT,), _f32)
    u, den_raw = _sc_aggr(xl, packed, lg, tmax, zrows, zvec)
    hidden_p = jnp.pad(hidden_states, ((0, NP - N), (0, 0)))
    q, h = _tc_post(u, den_raw.reshape(NC, NP, 1), hidden_p,
                    gbias.reshape(1, F),
                    Wih, bih.reshape(1, 3 * F), Whh, bhh.reshape(1, 3 * F),
                    Wq, bq.reshape(1, 16))
    return (q[:N], h[:N])


# speedup vs baseline: 15.2842x; 1.1223x over previous
import jax
import jax.numpy as jnp
from jax import lax
from jax.experimental import pallas as pl
from jax.experimental.pallas import tpu as pltpu
from jax.experimental.pallas import tpu_sc as plsc

N = 10000
E = 320000
F = 128
NC = 2
NS = 16
NW = NC * NS
L = 16
EPT = E // NW
B = 80
NB = EPT // B
NT = (NB - 1) // 2
NP = 10240
RPT = NP // NS
DPT = NP // NS
ZR = 80
ROWB = 1000
GRID = N // ROWB
ROWB2 = 1000
GRID2 = N // ROWB2

_f32 = jnp.float32
_i32 = jnp.int32


def _tc_pre_body(x_ref, w1_ref, b1_ref, w2_ref, b2_ref, wl_ref, wr_ref,
                 xl_ref, xr_ref):
    x = x_ref[...]
    h = jnp.maximum(jnp.dot(x, w1_ref[...], preferred_element_type=_f32)
                    + b1_ref[...], 0.0)
    h = jnp.maximum(jnp.dot(h, w2_ref[...], preferred_element_type=_f32)
                    + b2_ref[...], 0.0)
    xl_ref[...] = jnp.dot(h, wl_ref[...], preferred_element_type=_f32)
    xr_ref[...] = jnp.dot(h, wr_ref[...], preferred_element_type=_f32)


def _tc_pre(x, w1, b1, w2, b2, wl, wr):
    blk = lambda r, c: pl.BlockSpec((r, c), lambda i: (0, 0))
    return pl.pallas_call(
        _tc_pre_body,
        grid=(GRID,),
        in_specs=[
            pl.BlockSpec((ROWB, F), lambda i: (i, 0)),
            blk(F, F), blk(1, F), blk(F, F), blk(1, F), blk(F, F), blk(F, F),
        ],
        out_specs=[
            pl.BlockSpec((ROWB, F), lambda i: (i, 0)),
            pl.BlockSpec((ROWB, F), lambda i: (i, 0)),
        ],
        out_shape=[
            jax.ShapeDtypeStruct((N, F), _f32),
            jax.ShapeDtypeStruct((N, F), _f32),
        ],
    )(x, w1, b1, w2, b2, wl, wr)


def _sc_edge_body(xl_hbm, xr_hbm, att_hbm, packed_hbm, zrows_hbm, zvec_hbm,
                  u_hbm, den_hbm,
                  prow0, prow1, exrow0, exrow1, sbl0, sbl1, sbr0, sbr1, att_v,
                  u_sp, den_sp,
                  semP, semP2, semA0, semA1, semB0, semB1, semS):
    cid = lax.axis_index("c")
    sid = lax.axis_index("s")
    wid = sid * NC + cid
    for t in range(RPT // ZR):
        pltpu.sync_copy(zrows_hbm, u_sp.at[pl.ds(sid * RPT + t * ZR, ZR)])
    pltpu.sync_copy(zvec_hbm, den_sp.at[pl.ds(sid * DPT, DPT)])
    pltpu.sync_copy(att_hbm, att_v)
    att_chunks = [att_v[pl.ds(c * L, L)] for c in range(F // L)]
    lane = lax.broadcasted_iota(_i32, (L,), 0)
    perms = [jnp.bitwise_xor(lane, sh) for sh in (8, 4, 2, 1)]
    plsc.subcore_barrier()

    def issue_rows(j, prow, sem):
        pltpu.async_copy(packed_hbm.at[wid, j], prow, sem)

    def wait_rows(prow, sem):
        pltpu.make_async_copy(packed_hbm.at[wid, 0], prow, sem).wait()

    def issue_gathers(prow, lbuf, rbuf, semA, semB):
        pltpu.async_copy(xl_hbm.at[prow.at[0]], lbuf, semA)
        pltpu.async_copy(xr_hbm.at[prow.at[1]], rbuf, semB)

    def wait_gathers(lbuf, rbuf, semA, semB):
        pltpu.make_async_copy(xl_hbm.at[prow0.at[0]], lbuf, semA).wait()
        pltpu.make_async_copy(xr_hbm.at[prow0.at[1]], rbuf, semB).wait()

    def compute(prow, exrow, lbuf, rbuf):
        def gbody(g, _):
            exv16 = jnp.zeros((L,), _f32)
            for k in range(L):
                e = g * L + k
                ls = [lbuf[e, pl.ds(c * L, L)] for c in range(F // L)]
                rs = [rbuf[e, pl.ds(c * L, L)] for c in range(F // L)]
                accs = [jnp.zeros((L,), _f32) for _ in range(4)]
                for c in range(F // L):
                    v = ls[c] + rs[c]
                    accs[c % 4] = (accs[c % 4]
                                   + att_chunks[c] * jnp.maximum(v, 0.2 * v))
                acc = (accs[0] + accs[1]) + (accs[2] + accs[3])
                for p in perms:
                    acc = acc + acc.at[p].get(mode="promise_in_bounds",
                                              unique_indices=True)
                exv = jnp.exp(jnp.clip(acc, -60.0, 60.0))
                for c in range(F // L):
                    lbuf[e, pl.ds(c * L, L)] = ls[c] * exv
                exv16 = jnp.where(lane == k, exv, exv16)
            exrow[pl.ds(g * L, L)] = exv16
            return 0

        lax.fori_loop(0, B // L, gbody, 0)
        pltpu.async_copy(lbuf, u_sp.at[prow.at[1]], semS, add=True)
        pltpu.async_copy(exrow, den_sp.at[prow.at[1]], semS, add=True)

    def wait_scats(prow, exrow, lbuf):
        pltpu.make_async_copy(lbuf, u_sp.at[prow.at[1]], semS).wait()
        pltpu.make_async_copy(exrow, den_sp.at[prow.at[1]], semS).wait()

    pltpu.sync_copy(packed_hbm.at[wid, 0], prow0)
    issue_rows(1, prow1, semP)
    issue_gathers(prow0, sbl0, sbr0, semA0, semB0)

    def body(t, _):
        j0 = 2 * t
        j1 = j0 + 1
        wait_rows(prow1, semP)
        issue_gathers(prow1, sbl1, sbr1, semA1, semB1)
        wait_gathers(sbl0, sbr0, semA0, semB0)
        compute(prow0, exrow0, sbl0, sbr0)
        wait_gathers(sbl1, sbr1, semA1, semB1)
        compute(prow1, exrow1, sbl1, sbr1)
        wait_scats(prow0, exrow0, sbl0)
        issue_rows(j0 + 2, prow0, semP2)
        wait_scats(prow1, exrow1, sbl1)
        issue_rows(j1 + 2, prow1, semP)
        wait_rows(prow0, semP2)
        issue_gathers(prow0, sbl0, sbr0, semA0, semB0)
        return 0

    lax.fori_loop(0, NT, body, 0)
    wait_rows(prow1, semP)
    wait_gathers(sbl0, sbr0, semA0, semB0)
    compute(prow0, exrow0, sbl0, sbr0)
    wait_scats(prow0, exrow0, sbl0)

    plsc.subcore_barrier()
    pltpu.sync_copy(u_sp.at[pl.ds(sid * RPT, RPT)],
                    u_hbm.at[cid, pl.ds(sid * RPT, RPT)])
    pltpu.sync_copy(den_sp.at[pl.ds(sid * DPT, DPT)],
                    den_hbm.at[cid, pl.ds(sid * DPT, DPT)])


def _sc_edge(xl, xr, att, packed, zrows, zvec):
    mesh = plsc.VectorSubcoreMesh(core_axis_name="c", subcore_axis_name="s")
    fn = pl.kernel(
        _sc_edge_body,
        out_type=[
            jax.ShapeDtypeStruct((NC, NP, F), _f32),
            jax.ShapeDtypeStruct((NC, NP), _f32),
        ],
        mesh=mesh,
        scratch_types=[
            pltpu.VMEM((2, B), _i32),
            pltpu.VMEM((2, B), _i32),
            pltpu.VMEM((B,), _f32),
            pltpu.VMEM((B,), _f32),
            pltpu.VMEM((B, F), _f32),
            pltpu.VMEM((B, F), _f32),
            pltpu.VMEM((B, F), _f32),
            pltpu.VMEM((B, F), _f32),
            pltpu.VMEM((F,), _f32),
            pltpu.VMEM_SHARED((NP, F), _f32),
            pltpu.VMEM_SHARED((NP,), _f32),
            pltpu.SemaphoreType.DMA,
            pltpu.SemaphoreType.DMA,
            pltpu.SemaphoreType.DMA,
            pltpu.SemaphoreType.DMA,
            pltpu.SemaphoreType.DMA,
            pltpu.SemaphoreType.DMA,
            pltpu.SemaphoreType.DMA,
        ],
    )
    return fn(xl, xr, att, packed, zrows, zvec)


def _tc_post_body(u_ref, den_ref, h_ref, gb_ref, wih_ref, bih_ref,
                  whh_ref, bhh_ref, wq_ref, bq_ref, q_ref, hout_ref):
    u = u_ref[0] + u_ref[1]
    den = den_ref[0] + den_ref[1]
    gat = jnp.maximum(u / (den + 1e-16) + gb_ref[...], 0.0)
    h_in = h_ref[...]
    gi = jnp.dot(gat, wih_ref[...], preferred_element_type=_f32) + bih_ref[...]
    gh = jnp.dot(h_in, whh_ref[...], preferred_element_type=_f32) + bhh_ref[...]
    r = jax.nn.sigmoid(gi[:, :F] + gh[:, :F])
    z = jax.nn.sigmoid(gi[:, F:2 * F] + gh[:, F:2 * F])
    n = jnp.tanh(gi[:, 2 * F:] + r * gh[:, 2 * F:])
    h = (1.0 - z) * n + z * h_in
    q_ref[...] = jnp.dot(h, wq_ref[...], preferred_element_type=_f32) + bq_ref[...]
    hout_ref[...] = h


def _tc_post(u, den, hidden, gbias, wih, bih, whh, bhh, wq, bq):
    blk = lambda r, c: pl.BlockSpec((r, c), lambda i: (0, 0))
    return pl.pallas_call(
        _tc_post_body,
        grid=(GRID2,),
        in_specs=[
            pl.BlockSpec((NC, ROWB2, F), lambda i: (0, i, 0)),
            pl.BlockSpec((NC, ROWB2, 1), lambda i: (0, i, 0)),
            pl.BlockSpec((ROWB2, F), lambda i: (i, 0)),
            blk(1, F),
            blk(F, 3 * F), blk(1, 3 * F), blk(F, 3 * F), blk(1, 3 * F),
            blk(F, 16), blk(1, 16),
        ],
        out_specs=[
            pl.BlockSpec((ROWB2, 16), lambda i: (i, 0)),
            pl.BlockSpec((ROWB2, F), lambda i: (i, 0)),
        ],
        out_shape=[
            jax.ShapeDtypeStruct((N, 16), _f32),
            jax.ShapeDtypeStruct((N, F), _f32),
        ],
    )(u, den, hidden, gbias, wih, bih, whh, bhh, wq, bq)


def kernel(inputs, hidden_states, W1, b1, W2, b2, Wl, Wr, att, gbias,
           Wih, Whh, bih, bhh, Wq, bq, edge_index):
    xl, xr = _tc_pre(inputs, W1, b1.reshape(1, F), W2, b2.reshape(1, F),
                     Wl, Wr)
    src = edge_index[0].reshape(NW, NB, B)
    dst = edge_index[1].reshape(NW, NB, B)
    packed = jnp.pad(jnp.stack([src, dst], axis=2),
                     ((0, 0), (0, 1), (0, 0), (0, 0)))
    zrows = jnp.zeros((ZR, F), _f32)
    zvec = jnp.zeros((DPT,), _f32)
    u, den_raw = _sc_edge(xl, xr, att, packed, zrows, zvec)
    q, h = _tc_post(u, den_raw.reshape(NC, NP, 1), hidden_states,
                    gbias.reshape(1, F),
                    Wih, bih.reshape(1, 3 * F), Whh, bhh.reshape(1, 3 * F),
                    Wq, bq.reshape(1, 16))
    return (q, h)


# speedup vs baseline: 15.4987x; 1.0140x over previous
import jax
import jax.numpy as jnp
from jax import lax
from jax.experimental import pallas as pl
from jax.experimental.pallas import tpu as pltpu
from jax.experimental.pallas import tpu_sc as plsc

N = 10000
E = 320000
F = 128
NC = 2
NS = 16
NW = NC * NS
L = 16
EPT = E // NW
B = 80
NB = EPT // B
NT = (NB - 1) // 2
NP = 10240
RPT = NP // NS
DPT = NP // NS
ZR = 80
ROWB = 1000
GRID = N // ROWB
ROWB2 = 1000
GRID2 = N // ROWB2

_f32 = jnp.float32
_i32 = jnp.int32


def _tc_pre_body(x_ref, w1_ref, b1_ref, w2_ref, b2_ref, wl_ref, wr_ref,
                 xl_ref, xr_ref):
    x = x_ref[...]
    h = jnp.maximum(jnp.dot(x, w1_ref[...], preferred_element_type=_f32)
                    + b1_ref[...], 0.0)
    h = jnp.maximum(jnp.dot(h, w2_ref[...], preferred_element_type=_f32)
                    + b2_ref[...], 0.0)
    xl_ref[...] = jnp.dot(h, wl_ref[...], preferred_element_type=_f32)
    xr_ref[...] = jnp.dot(h, wr_ref[...], preferred_element_type=_f32)


def _tc_pre(x, w1, b1, w2, b2, wl, wr):
    blk = lambda r, c: pl.BlockSpec((r, c), lambda i: (0, 0))
    return pl.pallas_call(
        _tc_pre_body,
        grid=(GRID,),
        in_specs=[
            pl.BlockSpec((ROWB, F), lambda i: (i, 0)),
            blk(F, F), blk(1, F), blk(F, F), blk(1, F), blk(F, F), blk(F, F),
        ],
        out_specs=[
            pl.BlockSpec((ROWB, F), lambda i: (i, 0)),
            pl.BlockSpec((ROWB, F), lambda i: (i, 0)),
        ],
        out_shape=[
            jax.ShapeDtypeStruct((N, F), _f32),
            jax.ShapeDtypeStruct((N, F), _f32),
        ],
    )(x, w1, b1, w2, b2, wl, wr)


def _sc_edge_body(xl_hbm, xr_hbm, att_hbm, packed_hbm, zrows_hbm, zvec_hbm,
                  u_hbm, den_hbm,
                  prow0, prow1, exrow0, exrow1, sbl0, sbl1, sbr0, sbr1, att_v,
                  u_sp, den_sp,
                  semP, semP2, semA0, semA1, semB0, semB1, semS):
    cid = lax.axis_index("c")
    sid = lax.axis_index("s")
    wid = sid * NC + cid
    for t in range(RPT // ZR):
        pltpu.sync_copy(zrows_hbm, u_sp.at[pl.ds(sid * RPT + t * ZR, ZR)])
    pltpu.sync_copy(zvec_hbm, den_sp.at[pl.ds(sid * DPT, DPT)])
    pltpu.sync_copy(att_hbm, att_v)
    att_chunks = [att_v[pl.ds(c * L, L)] for c in range(F // L)]
    lane = lax.broadcasted_iota(_i32, (L,), 0)
    perms = [jnp.bitwise_xor(lane, sh) for sh in (8, 4, 2, 1)]
    plsc.subcore_barrier()

    def issue_rows(j, prow, sem):
        pltpu.async_copy(packed_hbm.at[wid, j], prow, sem)

    def wait_rows(prow, sem):
        pltpu.make_async_copy(packed_hbm.at[wid, 0], prow, sem).wait()

    def issue_gathers(prow, lbuf, rbuf, semA, semB):
        pltpu.async_copy(xl_hbm.at[prow.at[0]], lbuf, semA)
        pltpu.async_copy(xr_hbm.at[prow.at[1]], rbuf, semB)

    def wait_gathers(lbuf, rbuf, semA, semB):
        pltpu.make_async_copy(xl_hbm.at[prow0.at[0]], lbuf, semA).wait()
        pltpu.make_async_copy(xr_hbm.at[prow0.at[1]], rbuf, semB).wait()

    def compute(prow, exrow, lbuf, rbuf):
        def gbody(g, _):
            exv16 = jnp.zeros((L,), _f32)
            for k in range(L):
                e = g * L + k
                acc0 = jnp.zeros((L,), _f32)
                acc1 = jnp.zeros((L,), _f32)
                for c in range(F // L):
                    v = lbuf[e, pl.ds(c * L, L)] + rbuf[e, pl.ds(c * L, L)]
                    w = att_chunks[c] * jnp.maximum(v, 0.2 * v)
                    if c % 2:
                        acc1 = acc1 + w
                    else:
                        acc0 = acc0 + w
                acc = acc0 + acc1
                for p in perms:
                    acc = acc + acc.at[p].get(mode="promise_in_bounds",
                                              unique_indices=True)
                exv = jnp.exp(jnp.clip(acc, -60.0, 60.0))
                for c in range(F // L):
                    lbuf[e, pl.ds(c * L, L)] = lbuf[e, pl.ds(c * L, L)] * exv
                exv16 = jnp.where(lane == k, exv, exv16)
            exrow[pl.ds(g * L, L)] = exv16
            return 0

        lax.fori_loop(0, B // L, gbody, 0)
        pltpu.async_copy(lbuf, u_sp.at[prow.at[1]], semS, add=True)
        pltpu.async_copy(exrow, den_sp.at[prow.at[1]], semS, add=True)

    def wait_scats(prow, exrow, lbuf):
        pltpu.make_async_copy(lbuf, u_sp.at[prow.at[1]], semS).wait()
        pltpu.make_async_copy(exrow, den_sp.at[prow.at[1]], semS).wait()

    pltpu.sync_copy(packed_hbm.at[wid, 0], prow0)
    issue_rows(1, prow1, semP)
    issue_gathers(prow0, sbl0, sbr0, semA0, semB0)

    def body(t, _):
        j0 = 2 * t
        j1 = j0 + 1
        wait_rows(prow1, semP)
        issue_gathers(prow1, sbl1, sbr1, semA1, semB1)
        wait_gathers(sbl0, sbr0, semA0, semB0)
        compute(prow0, exrow0, sbl0, sbr0)
        wait_gathers(sbl1, sbr1, semA1, semB1)
        compute(prow1, exrow1, sbl1, sbr1)
        wait_scats(prow0, exrow0, sbl0)
        issue_rows(j0 + 2, prow0, semP2)
        wait_scats(prow1, exrow1, sbl1)
        issue_rows(j1 + 2, prow1, semP)
        wait_rows(prow0, semP2)
        issue_gathers(prow0, sbl0, sbr0, semA0, semB0)
        return 0

    lax.fori_loop(0, NT, body, 0)
    wait_rows(prow1, semP)
    wait_gathers(sbl0, sbr0, semA0, semB0)
    compute(prow0, exrow0, sbl0, sbr0)
    wait_scats(prow0, exrow0, sbl0)

    plsc.subcore_barrier()
    pltpu.sync_copy(u_sp.at[pl.ds(sid * RPT, RPT)],
                    u_hbm.at[cid, pl.ds(sid * RPT, RPT)])
    pltpu.sync_copy(den_sp.at[pl.ds(sid * DPT, DPT)],
                    den_hbm.at[cid, pl.ds(sid * DPT, DPT)])


def _sc_edge(xl, xr, att, packed, zrows, zvec):
    mesh = plsc.VectorSubcoreMesh(core_axis_name="c", subcore_axis_name="s")
    fn = pl.kernel(
        _sc_edge_body,
        out_type=[
            jax.ShapeDtypeStruct((NC, NP, F), _f32),
            jax.ShapeDtypeStruct((NC, NP), _f32),
        ],
        mesh=mesh,
        scratch_types=[
            pltpu.VMEM((2, B), _i32),
            pltpu.VMEM((2, B), _i32),
            pltpu.VMEM((B,), _f32),
            pltpu.VMEM((B,), _f32),
            pltpu.VMEM((B, F), _f32),
            pltpu.VMEM((B, F), _f32),
            pltpu.VMEM((B, F), _f32),
            pltpu.VMEM((B, F), _f32),
            pltpu.VMEM((F,), _f32),
            pltpu.VMEM_SHARED((NP, F), _f32),
            pltpu.VMEM_SHARED((NP,), _f32),
            pltpu.SemaphoreType.DMA,
            pltpu.SemaphoreType.DMA,
            pltpu.SemaphoreType.DMA,
            pltpu.SemaphoreType.DMA,
            pltpu.SemaphoreType.DMA,
            pltpu.SemaphoreType.DMA,
            pltpu.SemaphoreType.DMA,
        ],
    )
    return fn(xl, xr, att, packed, zrows, zvec)


def _tc_post_body(u_ref, den_ref, h_ref, gb_ref, wih_ref, bih_ref,
                  whh_ref, bhh_ref, wq_ref, bq_ref, q_ref, hout_ref):
    u = u_ref[0] + u_ref[1]
    den = den_ref[0] + den_ref[1]
    gat = jnp.maximum(u / (den + 1e-16) + gb_ref[...], 0.0)
    h_in = h_ref[...]
    gi = jnp.dot(gat, wih_ref[...], preferred_element_type=_f32) + bih_ref[...]
    gh = jnp.dot(h_in, whh_ref[...], preferred_element_type=_f32) + bhh_ref[...]
    r = jax.nn.sigmoid(gi[:, :F] + gh[:, :F])
    z = jax.nn.sigmoid(gi[:, F:2 * F] + gh[:, F:2 * F])
    n = jnp.tanh(gi[:, 2 * F:] + r * gh[:, 2 * F:])
    h = (1.0 - z) * n + z * h_in
    q_ref[...] = jnp.dot(h, wq_ref[...], preferred_element_type=_f32) + bq_ref[...]
    hout_ref[...] = h


def _tc_post(u, den, hidden, gbias, wih, bih, whh, bhh, wq, bq):
    blk = lambda r, c: pl.BlockSpec((r, c), lambda i: (0, 0))
    return pl.pallas_call(
        _tc_post_body,
        grid=(GRID2,),
        in_specs=[
            pl.BlockSpec((NC, ROWB2, F), lambda i: (0, i, 0)),
            pl.BlockSpec((NC, ROWB2, 1), lambda i: (0, i, 0)),
            pl.BlockSpec((ROWB2, F), lambda i: (i, 0)),
            blk(1, F),
            blk(F, 3 * F), blk(1, 3 * F), blk(F, 3 * F), blk(1, 3 * F),
            blk(F, 16), blk(1, 16),
        ],
        out_specs=[
            pl.BlockSpec((ROWB2, 16), lambda i: (i, 0)),
            pl.BlockSpec((ROWB2, F), lambda i: (i, 0)),
        ],
        out_shape=[
            jax.ShapeDtypeStruct((N, 16), _f32),
            jax.ShapeDtypeStruct((N, F), _f32),
        ],
    )(u, den, hidden, gbias, wih, bih, whh, bhh, wq, bq)


def kernel(inputs, hidden_states, W1, b1, W2, b2, Wl, Wr, att, gbias,
           Wih, Whh, bih, bhh, Wq, bq, edge_index):
    xl, xr = _tc_pre(inputs, W1, b1.reshape(1, F), W2, b2.reshape(1, F),
                     Wl, Wr)
    src = edge_index[0].reshape(NW, NB, B)
    dst = edge_index[1].reshape(NW, NB, B)
    packed = jnp.pad(jnp.stack([src, dst], axis=2),
                     ((0, 0), (0, 1), (0, 0), (0, 0)))
    zrows = jnp.zeros((ZR, F), _f32)
    zvec = jnp.zeros((DPT,), _f32)
    u, den_raw = _sc_edge(xl, xr, att, packed, zrows, zvec)
    q, h = _tc_post(u, den_raw.reshape(NC, NP, 1), hidden_states,
                    gbias.reshape(1, F),
                    Wih, bih.reshape(1, 3 * F), Whh, bhh.reshape(1, 3 * F),
                    Wq, bq.reshape(1, 16))
    return (q, h)


# speedup vs baseline: 15.7004x; 1.0130x over previous
import jax
import jax.numpy as jnp
from jax import lax
from jax.experimental import pallas as pl
from jax.experimental.pallas import tpu as pltpu
from jax.experimental.pallas import tpu_sc as plsc

N = 10000
E = 320000
F = 128
NC = 2
NS = 16
NW = NC * NS
L = 16
EPT = E // NW
B = 80
NB = EPT // B
NT = (NB - 1) // 2
NP = 10240
RPT = NP // NS
DPT = NP // NS
ZR = 80
ROWB = 1000
GRID = N // ROWB
ROWB2 = 1000
GRID2 = N // ROWB2

_f32 = jnp.float32
_i32 = jnp.int32


def _tc_pre_body(x_ref, w1_ref, b1_ref, w2_ref, b2_ref, wl_ref, wr_ref,
                 xl_ref, xr_ref):
    x = x_ref[...]
    h = jnp.maximum(jnp.dot(x, w1_ref[...], preferred_element_type=_f32)
                    + b1_ref[...], 0.0)
    h = jnp.maximum(jnp.dot(h, w2_ref[...], preferred_element_type=_f32)
                    + b2_ref[...], 0.0)
    xl_ref[...] = jnp.dot(h, wl_ref[...], preferred_element_type=_f32)
    xr_ref[...] = jnp.dot(h, wr_ref[...], preferred_element_type=_f32)


def _tc_pre(x, w1, b1, w2, b2, wl, wr):
    blk = lambda r, c: pl.BlockSpec((r, c), lambda i: (0, 0))
    return pl.pallas_call(
        _tc_pre_body,
        grid=(GRID,),
        in_specs=[
            pl.BlockSpec((ROWB, F), lambda i: (i, 0)),
            blk(F, F), blk(1, F), blk(F, F), blk(1, F), blk(F, F), blk(F, F),
        ],
        out_specs=[
            pl.BlockSpec((ROWB, F), lambda i: (i, 0)),
            pl.BlockSpec((ROWB, F), lambda i: (i, 0)),
        ],
        out_shape=[
            jax.ShapeDtypeStruct((N, F), _f32),
            jax.ShapeDtypeStruct((N, F), _f32),
        ],
    )(x, w1, b1, w2, b2, wl, wr)


def _sc_logits_body(xl_hbm, xr_hbm, att_hbm, src_hbm, dst_hbm,
                    logits_hbm, tmax_hbm,
                    src2, dst2, sbl0, sbl1, sbr0, sbr1, att_v, logit_v,
                    maxbuf, semA0, semA1, semB0, semB1):
    cid = lax.axis_index("c")
    sid = lax.axis_index("s")
    wid = sid * NC + cid
    pltpu.sync_copy(src_hbm.at[wid], src2)
    pltpu.sync_copy(dst_hbm.at[wid], dst2)
    pltpu.sync_copy(att_hbm, att_v)
    att_chunks = [att_v[pl.ds(c * L, L)] for c in range(F // L)]

    lane = lax.broadcasted_iota(_i32, (L,), 0)
    perms = [jnp.bitwise_xor(lane, sh) for sh in (8, 4, 2, 1)]

    def lanesum(v):
        for p in perms:
            v = v + v.at[p].get(mode="promise_in_bounds", unique_indices=True)
        return v

    def issue_xl(j, buf, sem):
        pltpu.async_copy(xl_hbm.at[src2.at[j]], buf, sem)

    def issue_xr(j, buf, sem):
        pltpu.async_copy(xr_hbm.at[dst2.at[j]], buf, sem)

    def wait_gather(buf, sem):
        pltpu.make_async_copy(xl_hbm.at[src2.at[0]], buf, sem).wait()

    def compute(j, lbuf, rbuf):
        def gbody(g, _):
            r = jnp.zeros((L,), _f32)
            for k in range(L):
                e = g * L + k
                accs = [jnp.zeros((L,), _f32) for _ in range(4)]
                for c in range(F // L):
                    v = lbuf[e, pl.ds(c * L, L)] + rbuf[e, pl.ds(c * L, L)]
                    accs[c % 4] = (accs[c % 4]
                                   + att_chunks[c] * jnp.maximum(v, 0.2 * v))
                acc = (accs[0] + accs[1]) + (accs[2] + accs[3])
                r = jnp.where(lane == k, lanesum(acc), r)
            logit_v[j, pl.ds(g * L, L)] = r
            return 0

        lax.fori_loop(0, B // L, gbody, 0)

    issue_xl(0, sbl0, semA0)
    issue_xr(0, sbr0, semB0)

    def body(t, _):
        j0 = 2 * t
        j1 = j0 + 1
        issue_xl(j1, sbl1, semA1)
        issue_xr(j1, sbr1, semB1)
        wait_gather(sbl0, semA0)
        wait_gather(sbr0, semB0)
        compute(j0, sbl0, sbr0)
        issue_xl(j0 + 2, sbl0, semA0)
        issue_xr(j0 + 2, sbr0, semB0)
        wait_gather(sbl1, semA1)
        wait_gather(sbr1, semB1)
        compute(j1, sbl1, sbr1)
        return 0

    lax.fori_loop(0, NT, body, 0)
    wait_gather(sbl0, semA0)
    wait_gather(sbr0, semB0)
    compute(NB - 1, sbl0, sbr0)

    def mbody(j, m):
        for c in range(B // L):
            m = jnp.maximum(m, logit_v[j, pl.ds(c * L, L)])
        return m

    m = lax.fori_loop(0, NB, mbody, jnp.full((L,), -3.4e38, _f32))
    maxbuf[...] = m
    pltpu.sync_copy(maxbuf, tmax_hbm.at[pl.ds(wid * L, L)])
    pltpu.sync_copy(logit_v, logits_hbm.at[wid])


def _sc_logits(xl, xr, att, src, dst):
    mesh = plsc.VectorSubcoreMesh(core_axis_name="c", subcore_axis_name="s")
    fn = pl.kernel(
        _sc_logits_body,
        out_type=[
            jax.ShapeDtypeStruct((NW, NB, B), _f32),
            jax.ShapeDtypeStruct((NW * L,), _f32),
        ],
        mesh=mesh,
        scratch_types=[
            pltpu.VMEM((NB, B), _i32),
            pltpu.VMEM((NB, B), _i32),
            pltpu.VMEM((B, F), _f32),
            pltpu.VMEM((B, F), _f32),
            pltpu.VMEM((B, F), _f32),
            pltpu.VMEM((B, F), _f32),
            pltpu.VMEM((F,), _f32),
            pltpu.VMEM((NB, B), _f32),
            pltpu.VMEM((L,), _f32),
            pltpu.SemaphoreType.DMA,
            pltpu.SemaphoreType.DMA,
            pltpu.SemaphoreType.DMA,
            pltpu.SemaphoreType.DMA,
        ],
    )
    return fn(xl, xr, att, src, dst)


def _sc_aggr_body(xl_hbm, packed_hbm, lg_hbm, tmax_hbm, zrows_hbm, zvec_hbm,
                  u_hbm, den_hbm,
                  prow0, prow1, lrow0, lrow1, exrow0, exrow1, rowbuf0,
                  rowbuf1, tmax_v, u_sp, den_sp,
                  semP, semP2, semL, semL2, semg0, semg1, semS):
    cid = lax.axis_index("c")
    sid = lax.axis_index("s")
    wid = sid * NC + cid
    for t in range(RPT // ZR):
        pltpu.sync_copy(zrows_hbm, u_sp.at[pl.ds(sid * RPT + t * ZR, ZR)])
    pltpu.sync_copy(zvec_hbm, den_sp.at[pl.ds(sid * DPT, DPT)])
    pltpu.sync_copy(tmax_hbm, tmax_v)

    def mb(i, m):
        return jnp.maximum(m, tmax_v[pl.ds(i * L, L)])

    m = lax.fori_loop(0, NW, mb, jnp.full((L,), -3.4e38, _f32))
    lane = lax.broadcasted_iota(_i32, (L,), 0)
    for sh in (8, 4, 2, 1):
        p = jnp.bitwise_xor(lane, sh)
        m = jnp.maximum(m, m.at[p].get(mode="promise_in_bounds",
                                       unique_indices=True))
    gmax = m
    plsc.subcore_barrier()

    def issue_rows(j, prow, sem):
        pltpu.async_copy(packed_hbm.at[wid, j], prow, sem)

    def wait_rows(prow, sem):
        pltpu.make_async_copy(packed_hbm.at[wid, 0], prow, sem).wait()

    def issue_lrow(j, lrow, sem):
        pltpu.async_copy(lg_hbm.at[wid, j], lrow, sem)

    def wait_lrow(lrow, sem):
        pltpu.make_async_copy(lg_hbm.at[wid, 0], lrow, sem).wait()

    def issue_gather(prow, buf, sem):
        pltpu.async_copy(xl_hbm.at[prow.at[0]], buf, sem)

    def wait_gather(buf, sem):
        pltpu.make_async_copy(xl_hbm.at[prow0.at[0]], buf, sem).wait()

    def compute(prow, lrow, exrow, buf):
        for c in range(B // L):
            lv = lrow[0, pl.ds(c * L, L)]
            exrow[pl.ds(c * L, L)] = jnp.exp(lv - gmax)

        def gbody(g, _):
            exvec = exrow[pl.ds(g * L, L)]
            for k in range(L):
                s = exvec[k]
                e = g * L + k
                for c in range(F // L):
                    buf[e, pl.ds(c * L, L)] = buf[e, pl.ds(c * L, L)] * s
            return 0

        lax.fori_loop(0, B // L, gbody, 0)
        pltpu.async_copy(buf, u_sp.at[prow.at[1]], semS, add=True)
        pltpu.async_copy(exrow, den_sp.at[prow.at[1]], semS, add=True)

    def wait_scats(prow, exrow, buf):
        pltpu.make_async_copy(buf, u_sp.at[prow.at[1]], semS).wait()
        pltpu.make_async_copy(exrow, den_sp.at[prow.at[1]], semS).wait()

    pltpu.sync_copy(packed_hbm.at[wid, 0], prow0)
    pltpu.sync_copy(lg_hbm.at[wid, 0], lrow0)
    issue_rows(1, prow1, semP)
    issue_lrow(1, lrow1, semL)
    issue_gather(prow0, rowbuf0, semg0)

    def body(t, _):
        j0 = 2 * t
        j1 = j0 + 1
        wait_rows(prow1, semP)
        issue_gather(prow1, rowbuf1, semg1)
        wait_gather(rowbuf0, semg0)
        compute(prow0, lrow0, exrow0, rowbuf0)
        issue_lrow(j0 + 2, lrow0, semL2)
        wait_gather(rowbuf1, semg1)
        wait_lrow(lrow1, semL)
        compute(prow1, lrow1, exrow1, rowbuf1)
        wait_scats(prow0, exrow0, rowbuf0)
        issue_rows(j0 + 2, prow0, semP2)
        wait_scats(prow1, exrow1, rowbuf1)
        issue_rows(j1 + 2, prow1, semP)
        issue_lrow(j1 + 2, lrow1, semL)
        wait_rows(prow0, semP2)
        wait_lrow(lrow0, semL2)
        issue_gather(prow0, rowbuf0, semg0)
        return 0

    lax.fori_loop(0, NT, body, 0)
    wait_rows(prow1, semP)
    wait_lrow(lrow1, semL)
    wait_gather(rowbuf0, semg0)
    compute(prow0, lrow0, exrow0, rowbuf0)
    wait_scats(prow0, exrow0, rowbuf0)

    plsc.subcore_barrier()
    pltpu.sync_copy(u_sp.at[pl.ds(sid * RPT, RPT)],
                    u_hbm.at[cid, pl.ds(sid * RPT, RPT)])
    pltpu.sync_copy(den_sp.at[pl.ds(sid * DPT, DPT)],
                    den_hbm.at[cid, pl.ds(sid * DPT, DPT)])


def _sc_aggr(xl, packed, lg, tmax, zrows, zvec):
    mesh = plsc.VectorSubcoreMesh(core_axis_name="c", subcore_axis_name="s")
    fn = pl.kernel(
        _sc_aggr_body,
        out_type=[
            jax.ShapeDtypeStruct((NC, NP, F), _f32),
            jax.ShapeDtypeStruct((NC, NP), _f32),
        ],
        mesh=mesh,
        scratch_types=[
            pltpu.VMEM((2, B), _i32),
            pltpu.VMEM((2, B), _i32),
            pltpu.VMEM((1, B), _f32),
            pltpu.VMEM((1, B), _f32),
            pltpu.VMEM((B,), _f32),
            pltpu.VMEM((B,), _f32),
            pltpu.VMEM((B, F), _f32),
            pltpu.VMEM((B, F), _f32),
            pltpu.VMEM((NW * L,), _f32),
            pltpu.VMEM_SHARED((NP, F), _f32),
            pltpu.VMEM_SHARED((NP,), _f32),
            pltpu.SemaphoreType.DMA,
            pltpu.SemaphoreType.DMA,
            pltpu.SemaphoreType.DMA,
            pltpu.SemaphoreType.DMA,
            pltpu.SemaphoreType.DMA,
            pltpu.SemaphoreType.DMA,
            pltpu.SemaphoreType.DMA,
        ],
    )
    return fn(xl, packed, lg, tmax, zrows, zvec)


def _tc_post_body(u_ref, den_ref, h_ref, gb_ref, wih_ref, bih_ref,
                  whh_ref, bhh_ref, wq_ref, bq_ref, q_ref, hout_ref):
    u = u_ref[0] + u_ref[1]
    den = den_ref[0] + den_ref[1]
    gat = jnp.maximum(u / (den + 1e-16) + gb_ref[...], 0.0)
    h_in = h_ref[...]
    gi = jnp.dot(gat, wih_ref[...], preferred_element_type=_f32) + bih_ref[...]
    gh = jnp.dot(h_in, whh_ref[...], preferred_element_type=_f32) + bhh_ref[...]
    r = jax.nn.sigmoid(gi[:, :F] + gh[:, :F])
    z = jax.nn.sigmoid(gi[:, F:2 * F] + gh[:, F:2 * F])
    n = jnp.tanh(gi[:, 2 * F:] + r * gh[:, 2 * F:])
    h = (1.0 - z) * n + z * h_in
    q_ref[...] = jnp.dot(h, wq_ref[...], preferred_element_type=_f32) + bq_ref[...]
    hout_ref[...] = h


def _tc_post(u, den, hidden, gbias, wih, bih, whh, bhh, wq, bq):
    blk = lambda r, c: pl.BlockSpec((r, c), lambda i: (0, 0))
    return pl.pallas_call(
        _tc_post_body,
        grid=(GRID2,),
        in_specs=[
            pl.BlockSpec((NC, ROWB2, F), lambda i: (0, i, 0)),
            pl.BlockSpec((NC, ROWB2, 1), lambda i: (0, i, 0)),
            pl.BlockSpec((ROWB2, F), lambda i: (i, 0)),
            blk(1, F),
            blk(F, 3 * F), blk(1, 3 * F), blk(F, 3 * F), blk(1, 3 * F),
            blk(F, 16), blk(1, 16),
        ],
        out_specs=[
            pl.BlockSpec((ROWB2, 16), lambda i: (i, 0)),
            pl.BlockSpec((ROWB2, F), lambda i: (i, 0)),
        ],
        out_shape=[
            jax.ShapeDtypeStruct((N, 16), _f32),
            jax.ShapeDtypeStruct((N, F), _f32),
        ],
    )(u, den, hidden, gbias, wih, bih, whh, bhh, wq, bq)


def kernel(inputs, hidden_states, W1, b1, W2, b2, Wl, Wr, att, gbias,
           Wih, Whh, bih, bhh, Wq, bq, edge_index):
    xl, xr = _tc_pre(inputs, W1, b1.reshape(1, F), W2, b2.reshape(1, F),
                     Wl, Wr)
    src = edge_index[0].reshape(NW, NB, B)
    dst = edge_index[1].reshape(NW, NB, B)
    logits, tmax = _sc_logits(xl, xr, att, src, dst)
    packed = jnp.pad(jnp.stack([src, dst], axis=2),
                     ((0, 0), (0, 1), (0, 0), (0, 0)))
    lg = jnp.pad(logits, ((0, 0), (0, 1), (0, 0))).reshape(NW, NB + 1, 1, B)
    zrows = jnp.zeros((ZR, F), _f32)
    zvec = jnp.zeros((DPT,), _f32)
    u, den_raw = _sc_aggr(xl, packed, lg, tmax, zrows, zvec)
    q, h = _tc_post(u, den_raw.reshape(NC, NP, 1), hidden_states,
                    gbias.reshape(1, F),
                    Wih, bih.reshape(1, 3 * F), Whh, bhh.reshape(1, 3 * F),
                    Wq, bq.reshape(1, 16))
    return (q, h)
